# Initial kernel scaffold; baseline (speedup 1.0000x reference)
import jax
import jax.numpy as jnp
from jax.experimental import pallas as pl


def kernel(R_complex, W, b):
    raise NotImplementedError("write your pallas kernel here")


# speedup vs baseline: 1.0340x; 1.0340x over previous
import functools

import jax
import jax.numpy as jnp
from jax import lax
from jax.experimental import pallas as pl
from jax.experimental.pallas import tpu as pltpu
from jax.experimental.pallas import tpu_sc as plsc

N = 128
ND = 2 * N - 1
FEAT = 2 * ND
FPAD = 512
OUT = 256
ROW = 2 * N
WORDS = N * ROW

NC = 2
NS = 16
NW = NC * NS


def _sc_pool(r_flat):
    B = r_flat.shape[0]
    per_w = B // NW
    mesh = plsc.VectorSubcoreMesh(core_axis_name="c", subcore_axis_name="s")

    @functools.partial(
        pl.kernel,
        mesh=mesh,
        out_type=jax.ShapeDtypeStruct((B, FPAD), jnp.float32),
        scratch_types=[
            pltpu.VMEM((WORDS,), jnp.float32),
            pltpu.VMEM((FPAD + 16,), jnp.float32),
        ],
    )
    def k(r_hbm, out_hbm, buf, acc):
        wid = lax.axis_index("s") * NC + lax.axis_index("c")
        base = wid * per_w

        def body(t, carry):
            pltpu.sync_copy(r_hbm.at[base + t], buf)
            zero = jnp.zeros((16,), jnp.float32)
            for cidx in range((FPAD + 16) // 16):
                acc[pl.ds(cidx * 16, 16)] = zero

            def row_body(i, c2):
                for kk in range(16):
                    v = buf[pl.ds(i * ROW + kk * 16, 16)]
                    plsc.addupdate(acc.at[pl.ds(2 * i + kk * 16, 16)], v)
                return c2

            lax.fori_loop(0, N, row_body, 0)
            pltpu.sync_copy(acc.at[pl.ds(0, FPAD)], out_hbm.at[base + t])
            return carry

        lax.fori_loop(0, per_w, body, 0)

    return k(r_flat)


def _tc_proj(sums, w_eff, bias):
    B = sums.shape[0]
    blk = 256

    def mm_kernel(s_ref, w_ref, b_ref, o_ref):
        o_ref[...] = (
            jnp.dot(
                s_ref[...],
                w_ref[...],
                preferred_element_type=jnp.float32,
                precision=jax.lax.Precision.HIGHEST,
            )
            + b_ref[...]
        )

    return pl.pallas_call(
        mm_kernel,
        grid=(B // blk,),
        in_specs=[
            pl.BlockSpec((blk, FPAD), lambda i: (i, 0)),
            pl.BlockSpec((FPAD, OUT), lambda i: (0, 0)),
            pl.BlockSpec((1, OUT), lambda i: (0, 0)),
        ],
        out_specs=pl.BlockSpec((blk, OUT), lambda i: (i, 0)),
        out_shape=jax.ShapeDtypeStruct((B, OUT), jnp.float32),
    )(sums, w_eff, bias.reshape(1, OUT))


def kernel(R_complex, W, b):
    B = R_complex.shape[0]
    r_flat = R_complex.reshape(B, WORDS)
    d = jnp.arange(ND)
    counts = (N - jnp.abs(d - (N - 1))).astype(jnp.float32)
    w_r = W[:ND] / counts[:, None]
    w_i = W[ND:] / counts[:, None]
    w_il = jnp.stack([w_r, w_i], axis=1).reshape(FEAT, OUT)
    w_eff = jnp.concatenate([w_il, jnp.zeros((2, OUT), W.dtype)], axis=0)
    sums = _sc_pool(r_flat)
    return _tc_proj(sums, w_eff, b)


# speedup vs baseline: 6.8226x; 6.5984x over previous
import functools

import jax
import jax.numpy as jnp
from jax import lax
from jax.experimental import pallas as pl
from jax.experimental.pallas import tpu as pltpu
from jax.experimental.pallas import tpu_sc as plsc

N = 128
ND = 2 * N - 1
FPAD = 512
OUT = 256
ROWS = 2 * N

NC = 2
NS = 16
NW = NC * NS


def _sc_pool(r3):
    B = r3.shape[0]
    per_w = B // NW
    mesh = plsc.VectorSubcoreMesh(core_axis_name="c", subcore_axis_name="s")

    @functools.partial(
        pl.kernel,
        mesh=mesh,
        out_type=jax.ShapeDtypeStruct((B, FPAD), jnp.float32),
        scratch_types=[
            pltpu.VMEM((ROWS, N), jnp.float32),
            pltpu.VMEM((ROWS, N), jnp.float32),
            pltpu.VMEM((FPAD,), jnp.float32),
            pltpu.VMEM((FPAD,), jnp.float32),
            pltpu.SemaphoreType.DMA,
            pltpu.SemaphoreType.DMA,
            pltpu.SemaphoreType.DMA,
            pltpu.SemaphoreType.DMA,
        ],
    )
    def k(r_hbm, out_hbm, buf_a, buf_b, acc_a, acc_b, sem_a, sem_b, sem_oa, sem_ob):
        wid = lax.axis_index("s") * NC + lax.axis_index("c")
        base = wid * per_w
        zero = jnp.zeros((16,), jnp.float32)

        pltpu.make_async_copy(r_hbm.at[base], buf_a, sem_a).start()
        pltpu.make_async_copy(r_hbm.at[base + 1], buf_b, sem_b).start()

        def process(t, p, buf, acc, sem, sem_o):
            @pl.when(p > 0)
            def _():
                pltpu.make_async_copy(acc, out_hbm.at[base + t], sem_o).wait()

            pltpu.make_async_copy(r_hbm.at[base + t], buf, sem).wait()
            for cidx in range(FPAD // 16):
                acc[pl.ds(cidx * 16, 16)] = zero

            def row_body(i, c2):
                vs = [buf[2 * i + c, pl.ds(kk * 16, 16)]
                      for c in range(2) for kk in range(8)]
                for c in range(2):
                    for kk in range(8):
                        plsc.addupdate(
                            acc.at[pl.ds(256 * c + i + kk * 16, 16)],
                            vs[c * 8 + kk],
                        )
                return c2

            lax.fori_loop(0, N, row_body, 0)

            @pl.when(t + 2 < per_w)
            def _():
                pltpu.make_async_copy(r_hbm.at[base + t + 2], buf, sem).start()

            pltpu.make_async_copy(acc, out_hbm.at[base + t], sem_o).start()

        def pair_body(p, carry):
            t0 = 2 * p
            process(t0, p, buf_a, acc_a, sem_a, sem_oa)
            process(t0 + 1, p, buf_b, acc_b, sem_b, sem_ob)
            return carry

        lax.fori_loop(0, per_w // 2, pair_body, 0)
        pltpu.make_async_copy(acc_a, out_hbm.at[base + per_w - 2], sem_oa).wait()
        pltpu.make_async_copy(acc_b, out_hbm.at[base + per_w - 1], sem_ob).wait()

    return k(r3)


def _tc_proj(sums, w_eff, bias):
    B = sums.shape[0]
    blk = 256

    def mm_kernel(s_ref, w_ref, b_ref, o_ref):
        o_ref[...] = (
            jnp.dot(
                s_ref[...],
                w_ref[...],
                preferred_element_type=jnp.float32,
                precision=jax.lax.Precision.HIGHEST,
            )
            + b_ref[...]
        )

    return pl.pallas_call(
        mm_kernel,
        grid=(B // blk,),
        in_specs=[
            pl.BlockSpec((blk, FPAD), lambda i: (i, 0)),
            pl.BlockSpec((FPAD, OUT), lambda i: (0, 0)),
            pl.BlockSpec((1, OUT), lambda i: (0, 0)),
        ],
        out_specs=pl.BlockSpec((blk, OUT), lambda i: (i, 0)),
        out_shape=jax.ShapeDtypeStruct((B, OUT), jnp.float32),
    )(sums, w_eff, bias.reshape(1, OUT))


def kernel(R_complex, W, b):
    B = R_complex.shape[0]
    r3 = jnp.transpose(R_complex, (0, 1, 3, 2)).reshape(B, ROWS, N)
    d = jnp.arange(ND)
    counts = (N - jnp.abs(d - (N - 1))).astype(jnp.float32)
    w_r = W[:ND] / counts[:, None]
    w_i = W[ND:] / counts[:, None]
    zrow = jnp.zeros((1, OUT), W.dtype)
    w_eff = jnp.concatenate([w_r, zrow, w_i, zrow], axis=0)
    sums = _sc_pool(r3)
    return _tc_proj(sums, w_eff, b)
